# baseline (speedup 1.0000x reference)
import functools

import jax
import jax.numpy as jnp
from jax import lax
from jax.experimental import pallas as pl
from jax.experimental.pallas import tpu as pltpu
from jax.experimental.pallas import tpu_sc as plsc

N_PERTS = 1000000
N_LATENT = 64
B = 16384
M = 2
PADDING_IDX = 0

NC = 2
NS = 16
NW = NC * NS
PER_W = B // NW
K = PER_W * M
CH = 128
NCH = K // CH
L = 16

GC = 128
NGC = K // GC
NBUF = 2
UNROLL = 4


def _sc_kernel(perts_hbm, dos_hbm, emb_hbm, beta_hbm, bias_hbm, out_hbm,
               idx_v, qidx_v, hidx_v, dos_v, betag_v, biasg_v, coeff_v,
               out_v, rows_v, sems):
    wid = lax.axis_index("s") * NC + lax.axis_index("c")
    row0 = wid * NCH

    pltpu.sync_copy(perts_hbm.at[pl.ds(row0, NCH)], idx_v)
    pltpu.sync_copy(dos_hbm.at[pl.ds(row0, NCH)], dos_v)

    copies = []
    for j in range(NCH):
        idx_j = idx_v.at[j]
        copies.append(pltpu.async_copy(beta_hbm.at[idx_j], betag_v.at[j],
                                       sems.at[2]))
        copies.append(pltpu.async_copy(bias_hbm.at[idx_j], biasg_v.at[j],
                                       sems.at[2]))
    for j in range(NCH):
        for oi in range(CH // L):
            o = oi * L
            p = idx_v[j, pl.ds(o, L)]
            f = (j * (CH // L) + oi) * L
            qidx_v[pl.ds(f, L)] = p >> 1
            hidx_v[pl.ds(f, L)] = (p & 1) << 6
    for c in copies:
        c.wait()

    def fire(goff, buf):
        return pltpu.async_copy(
            emb_hbm.at[qidx_v.at[pl.ds(goff * GC, GC)]], rows_v.at[buf],
            sems.at[buf])

    def drain(buf):
        pltpu.make_async_copy(
            emb_hbm.at[qidx_v.at[pl.ds(0, GC)]], rows_v.at[buf],
            sems.at[buf]).wait()

    fire(0, 0)
    fire(1, 1)

    for j in range(NCH):
        for oi in range(CH // L):
            o = oi * L
            d = dos_v[j, pl.ds(o, L)]
            bg = betag_v[j, pl.ds(o, L)]
            hg = biasg_v[j, pl.ds(o, L)]
            p = idx_v[j, pl.ds(o, L)]
            t = d / (d + 2.0)
            t2 = t * t
            l1p = 2.0 * t * (1.0 + t2 * (1.0 / 3.0 + t2 * (
                0.2 + t2 * (1.0 / 7.0 + t2 * (1.0 / 9.0)))))
            z = l1p * bg + hg
            s = 1.0 / (1.0 + jnp.exp(-z))
            s0 = 1.0 / (1.0 + jnp.exp(-hg))
            c = jnp.where(p == PADDING_IDX, 0.0, s - s0)
            coeff_v[pl.ds((j * (CH // L) + oi) * L, L)] = c

    def combine(g, buf):
        def pair_body(ii, _):
            for u in range(UNROLL):
                jj = ii * UNROLL + u
                k = g * GC + 2 * jj
                cv = coeff_v[pl.ds(k, L)]
                hv = hidx_v[pl.ds(k, L)]
                c0, c1 = cv[0], cv[1]
                h0, h1 = hv[0], hv[1]
                r0 = rows_v.at[buf].at[2 * jj]
                r1 = rows_v.at[buf].at[2 * jj + 1]
                o = out_v.at[lax.div(k, 2)]
                for q in range(N_LATENT // L):
                    t0 = r0[pl.ds(h0 + q * L, L)]
                    t1 = r1[pl.ds(h1 + q * L, L)]
                    o[pl.ds(q * L, L)] = c0 * t0 + c1 * t1
            return 0
        lax.fori_loop(0, GC // 2 // UNROLL, pair_body, 0)

    def outer(g2, _):
        g0 = g2 * NBUF
        for b in range(NBUF):
            g = g0 + b
            drain(b)
            combine(g, b)
            fire(jnp.minimum(g + NBUF, NGC - 1), b)
        return 0

    lax.fori_loop(0, NGC // NBUF, outer, 0)
    for b in range(NBUF):
        drain(b)

    pltpu.sync_copy(out_v, out_hbm.at[pl.ds(wid * PER_W, PER_W)])


@jax.jit
def kernel(perts, dosages, embedding, beta, bias):
    perts2d = perts.astype(jnp.int32).reshape(B * M // CH, CH)
    dos2d = dosages.astype(jnp.float32).reshape(B * M // CH, CH)
    beta_f = beta.reshape(N_PERTS)
    bias_f = bias.reshape(N_PERTS)
    emb128 = embedding.reshape(N_PERTS // 2, 2 * N_LATENT)

    mesh = plsc.VectorSubcoreMesh(core_axis_name="c", subcore_axis_name="s")
    fn = functools.partial(
        pl.kernel,
        mesh=mesh,
        out_type=jax.ShapeDtypeStruct((B, N_LATENT), jnp.float32),
        scratch_types=[
            pltpu.VMEM((NCH, CH), jnp.int32),
            pltpu.VMEM((K,), jnp.int32),
            pltpu.VMEM((K + L,), jnp.int32),
            pltpu.VMEM((NCH, CH), jnp.float32),
            pltpu.VMEM((NCH, CH), jnp.float32),
            pltpu.VMEM((NCH, CH), jnp.float32),
            pltpu.VMEM((K + L,), jnp.float32),
            pltpu.VMEM((PER_W, N_LATENT), jnp.float32),
            pltpu.VMEM((NBUF, GC, 2 * N_LATENT), jnp.float32),
            pltpu.SemaphoreType.DMA((3,)),
        ],
    )(_sc_kernel)
    return fn(perts2d, dos2d, emb128, beta_f, bias_f)


# speedup vs baseline: 1.2060x; 1.2060x over previous
import functools

import jax
import jax.numpy as jnp
from jax import lax
from jax.experimental import pallas as pl
from jax.experimental.pallas import tpu as pltpu
from jax.experimental.pallas import tpu_sc as plsc

N_PERTS = 1000000
N_LATENT = 64
B = 16384
M = 2
PADDING_IDX = 0

NC = 2
NS = 16
NW = NC * NS
PER_W = B // NW
K = PER_W * M
CH = 128
NCH = K // CH
L = 16

NSLOT = 16
NROUND = PER_W // NSLOT - 1


def _sc_kernel(perts_hbm, dos_hbm, emb_hbm, beta_hbm, bias_hbm, out_hbm,
               idx_v, dos_v, betag_v, biasg_v, coeff_v, out_v, rows_v,
               gsem, sems):
    wid = lax.axis_index("s") * NC + lax.axis_index("c")
    base0 = wid * K

    pltpu.sync_copy(perts_hbm.at[pl.ds(base0, K)], idx_v.at[pl.ds(0, K)])
    pltpu.sync_copy(dos_hbm.at[pl.ds(base0, K)], dos_v)

    copies = []
    for j in range(NCH):
        idx_j = idx_v.at[pl.ds(j * CH, CH)]
        copies.append(pltpu.async_copy(
            beta_hbm.at[idx_j], betag_v.at[pl.ds(j * CH, CH)], gsem))
        copies.append(pltpu.async_copy(
            bias_hbm.at[idx_j], biasg_v.at[pl.ds(j * CH, CH)], gsem))
    for c in copies:
        c.wait()

    def fire(i, s):
        pv = idx_v[pl.ds(2 * i, L)]
        o0 = pl.multiple_of((pv[0] >> 3) * 8, 8)
        o1 = pl.multiple_of((pv[1] >> 3) * 8, 8)
        pltpu.async_copy(emb_hbm.at[pl.ds(o0, 8)], rows_v.at[s].at[0],
                         sems.at[s])
        pltpu.async_copy(emb_hbm.at[pl.ds(o1, 8)], rows_v.at[s].at[1],
                         sems.at[s])

    def drain(s):
        pltpu.make_async_copy(emb_hbm.at[pl.ds(0, 8)], rows_v.at[s].at[0],
                              sems.at[s]).wait()
        pltpu.make_async_copy(emb_hbm.at[pl.ds(0, 8)], rows_v.at[s].at[1],
                              sems.at[s]).wait()

    for s in range(NSLOT):
        fire(s, s)

    for g in range(K // L):
        o = g * L
        d = dos_v[pl.ds(o, L)]
        bg = betag_v[pl.ds(o, L)]
        hg = biasg_v[pl.ds(o, L)]
        p = idx_v[pl.ds(o, L)]
        t = d / (d + 2.0)
        t2 = t * t
        l1p = 2.0 * t * (1.0 + t2 * (1.0 / 3.0 + t2 * (
            0.2 + t2 * (1.0 / 7.0 + t2 * (1.0 / 9.0)))))
        z = l1p * bg + hg
        sg = 1.0 / (1.0 + jnp.exp(-z))
        s0 = 1.0 / (1.0 + jnp.exp(-hg))
        c = jnp.where(p == PADDING_IDX, 0.0, sg - s0)
        coeff_v[pl.ds(o, L)] = c

    def combine(i, s):
        cv = coeff_v[pl.ds(2 * i, L)]
        pv = idx_v[pl.ds(2 * i, L)]
        c0, c1 = cv[0], cv[1]
        r0 = rows_v.at[s].at[0].at[pv[0] & 7]
        r1 = rows_v.at[s].at[1].at[pv[1] & 7]
        o = out_v.at[i]
        for q in range(N_LATENT // L):
            sl = pl.ds(q * L, L)
            o[sl] = c0 * r0[sl] + c1 * r1[sl]

    def ring_round(c, _):
        base = c * NSLOT
        for s in range(NSLOT):
            i = base + s
            drain(s)
            combine(i, s)
            fire(i + NSLOT, s)
        return 0

    lax.fori_loop(0, NROUND, ring_round, 0)
    tail = NROUND * NSLOT
    for s in range(NSLOT):
        drain(s)
        combine(tail + s, s)

    pltpu.sync_copy(out_v, out_hbm.at[pl.ds(wid * PER_W, PER_W)])


@jax.jit
def kernel(perts, dosages, embedding, beta, bias):
    perts_f = perts.astype(jnp.int32).reshape(B * M)
    dos_f = dosages.astype(jnp.float32).reshape(B * M)
    beta_f = beta.reshape(N_PERTS)
    bias_f = bias.reshape(N_PERTS)

    mesh = plsc.VectorSubcoreMesh(core_axis_name="c", subcore_axis_name="s")
    fn = functools.partial(
        pl.kernel,
        mesh=mesh,
        out_type=jax.ShapeDtypeStruct((B, N_LATENT), jnp.float32),
        scratch_types=[
            pltpu.VMEM((K + L,), jnp.int32),
            pltpu.VMEM((K,), jnp.float32),
            pltpu.VMEM((K,), jnp.float32),
            pltpu.VMEM((K,), jnp.float32),
            pltpu.VMEM((K + L,), jnp.float32),
            pltpu.VMEM((PER_W, N_LATENT), jnp.float32),
            pltpu.VMEM((NSLOT, M, 8, N_LATENT), jnp.float32),
            pltpu.SemaphoreType.DMA,
            pltpu.SemaphoreType.DMA((NSLOT,)),
        ],
    )(_sc_kernel)
    return fn(perts_f, dos_f, embedding, beta_f, bias_f)
